# baseline (speedup 1.0000x reference)
import functools
import math

import jax
import jax.numpy as jnp
from jax import lax
from jax.experimental import pallas as pl
from jax.experimental.pallas import tpu as pltpu
from jax.experimental.pallas import tpu_sc as plsc


@functools.lru_cache(maxsize=None)
def _build(B, L, D, maxlen):
    info = plsc.get_sparse_core_info()
    NC, NS, LANES = info.num_cores, info.num_subcores, info.num_lanes
    NW = NC * NS
    assert B % NW == 0
    seqs_per_w = B // NW
    G = 4
    assert seqs_per_w % G == 0
    n_chunks = seqs_per_w // G
    ROWS = G * L
    assert (ROWS * D) % 8 == 0 and D % LANES == 0
    NJ = D // LANES
    scale = math.sqrt(D)

    mesh = plsc.VectorSubcoreMesh(core_axis_name="c", subcore_axis_name="s")

    @functools.partial(
        pl.kernel,
        out_type=jax.ShapeDtypeStruct((B * L, D), jnp.float32),
        mesh=mesh,
        compiler_params=pltpu.CompilerParams(use_tc_tiling_on_sc=False),
        scratch_types=[
            pltpu.VMEM((ROWS,), jnp.int32),
            pltpu.VMEM((ROWS, D), jnp.float32),
            pltpu.VMEM((L, D), jnp.float32),
            pltpu.SemaphoreType.DMA,
        ],
    )
    def emb(x_hbm, tok_hbm, pos_hbm, out_hbm, idx_v, rows_v, pos_v, sem):
        wid = lax.axis_index("s") * NC + lax.axis_index("c")
        pltpu.sync_copy(pos_hbm.at[pl.ds(0, L)], pos_v)

        def chunk_body(c, carry):
            base = (wid * n_chunks + c) * ROWS
            base = pl.multiple_of(base, 8)
            pltpu.sync_copy(x_hbm.at[pl.ds(base, ROWS)], idx_v)
            pltpu.async_copy(tok_hbm.at[idx_v], rows_v, sem).wait()

            def l_body(l, lc):
                p = [pos_v[l, pl.ds(LANES * j, LANES)] for j in range(NJ)]
                for g in range(G):
                    r = g * L + l
                    for j in range(NJ):
                        rows_v[r, pl.ds(LANES * j, LANES)] = (
                            rows_v[r, pl.ds(LANES * j, LANES)] + p[j]
                        ) * scale
                return lc

            lax.fori_loop(0, L, l_body, 0)
            pltpu.sync_copy(rows_v, out_hbm.at[pl.ds(base, ROWS)])
            return carry

        lax.fori_loop(0, n_chunks, chunk_body, 0)

    return emb


def kernel(x, token_table, pos_table):
    B, L = x.shape
    D = token_table.shape[1]
    emb = _build(B, L, D, pos_table.shape[0])
    x_flat = x.reshape(B * L).astype(jnp.int32)
    out = emb(x_flat, token_table, pos_table)
    return out.reshape(B, L, D)


# speedup vs baseline: 1.2240x; 1.2240x over previous
import functools
import math

import jax
import jax.numpy as jnp
from jax import lax
from jax.experimental import pallas as pl
from jax.experimental.pallas import tpu as pltpu
from jax.experimental.pallas import tpu_sc as plsc


@functools.lru_cache(maxsize=None)
def _build(B, L, D, V, maxlen):
    info = plsc.get_sparse_core_info()
    NC, NS, LANES = info.num_cores, info.num_subcores, info.num_lanes
    NW = NC * NS
    assert B % NW == 0 and V % (2 * NW) == 0 and D % LANES == 0
    scale = math.sqrt(D)
    NJ = D // LANES

    mesh = plsc.VectorSubcoreMesh(core_axis_name="c", subcore_axis_name="s")

    CH_A = 400
    n_chunks_a = V // CH_A
    trips_a = -(-n_chunks_a // NW)

    @functools.partial(
        pl.kernel,
        out_type=jax.ShapeDtypeStruct((V, 2 * D), jnp.float32),
        mesh=mesh,
        scratch_types=[
            pltpu.VMEM((CH_A, D), jnp.float32),
        ],
    )
    def mirror(tok_hbm, dense_hbm, buf_a):
        wid = lax.axis_index("s") * NC + lax.axis_index("c")

        def chunk_body(t, carry):
            cg = t * NW + wid

            @pl.when(cg < n_chunks_a)
            def _():
                pltpu.sync_copy(tok_hbm.at[pl.ds(cg * CH_A, CH_A)], buf_a)
                pltpu.sync_copy(
                    buf_a,
                    dense_hbm.at[pl.ds(cg * CH_A, CH_A), pl.ds(0, D)],
                )

            return carry

        lax.fori_loop(0, trips_a, chunk_body, 0)

    seqs_w = B // NW
    G = 2
    n_chunks_b = seqs_w // G
    ROWS = G * L

    @functools.partial(
        pl.kernel,
        out_type=jax.ShapeDtypeStruct((B * L, 2 * D), jnp.float32),
        mesh=mesh,
        compiler_params=pltpu.CompilerParams(use_tc_tiling_on_sc=False),
        scratch_types=[
            pltpu.VMEM((ROWS,), jnp.int32),
            pltpu.VMEM((ROWS, D), jnp.float32),
            pltpu.VMEM((L, D), jnp.float32),
            pltpu.SemaphoreType.DMA,
        ],
    )
    def gather(x2_hbm, tok_hbm, pos_hbm, out_hbm, idx_v, rows_v, pos_v, sem):
        wid = lax.axis_index("s") * NC + lax.axis_index("c")
        pltpu.sync_copy(pos_hbm.at[pl.ds(0, L)], pos_v)

        def chunk_body(c, carry):
            base = (wid * n_chunks_b + c) * ROWS
            base = pl.multiple_of(base, 8)
            pltpu.sync_copy(x2_hbm.at[pl.ds(base, ROWS)], idx_v)
            pltpu.async_copy(tok_hbm.at[idx_v], rows_v, sem).wait()

            def l_body(l, lc):
                p = [pos_v[l, pl.ds(LANES * j, LANES)] for j in range(NJ)]
                for g in range(G):
                    r = g * L + l
                    for j in range(NJ):
                        rows_v[r, pl.ds(LANES * j, LANES)] = (
                            rows_v[r, pl.ds(LANES * j, LANES)] + p[j]
                        ) * scale
                return lc

            lax.fori_loop(0, L, l_body, 0)
            pltpu.sync_copy(
                rows_v, out_hbm.at[pl.ds(base, ROWS), pl.ds(0, D)]
            )
            return carry

        lax.fori_loop(0, n_chunks_b, chunk_body, 0)

    def run(x, token_table, pos_table):
        x_flat = x.reshape(B * L).astype(jnp.int32)
        gathered = gather(x_flat, token_table, pos_table)
        return gathered.reshape(B, L, 2 * D)[:, :, :D]

    return run


def kernel(x, token_table, pos_table):
    B, L = x.shape
    V, D = token_table.shape
    run = _build(B, L, D, V, pos_table.shape[0])
    return run(x, token_table, pos_table)
